# baseline (speedup 1.0000x reference)
import functools

import jax
import jax.numpy as jnp
from jax import lax
from jax.experimental import pallas as pl
from jax.experimental.pallas import tpu as pltpu
from jax.experimental.pallas import tpu_sc as plsc

_C = 1.0
_NS = 16
_NC = 2


def _pre_body(x_ref, w2_ref, xt_ref, b2_ref):
    x = x_ref[...]
    nrm = jnp.maximum(jnp.sqrt(jnp.sum(x * x, axis=1, keepdims=True)), 1e-15)
    cn = jnp.clip(nrm, -1.0 + 1e-7, 1.0 - 1e-7)
    artanh = 0.5 * (jnp.log1p(cn) - jnp.log1p(-cn))
    xt = (artanh / nrm) * x
    xt_ref[...] = xt
    b2_ref[...] = jnp.sum(xt * w2_ref[...], axis=1, keepdims=True)


def _pre(x, w2row, bn):
    n, d = x.shape
    grid = (n // bn,)
    return pl.pallas_call(
        _pre_body,
        grid=grid,
        in_specs=[
            pl.BlockSpec((bn, d), lambda i: (i, 0)),
            pl.BlockSpec((1, d), lambda i: (0, 0)),
        ],
        out_specs=[
            pl.BlockSpec((bn, d), lambda i: (i, 0)),
            pl.BlockSpec((bn, 1), lambda i: (i, 0)),
        ],
        out_shape=[
            jax.ShapeDtypeStruct((n, d), jnp.float32),
            jax.ShapeDtypeStruct((n, 1), jnp.float32),
        ],
    )(x, w2row)


def _sc_body(np_, e, d, chunk,
             ea, srci, dsti, b2h, z2, z1, ones_in,
             m0o, m1o, cso, cdo, t1ao, t1bo,
             m0sh, m1sh, ssh_a, ssh_b, b2sh,
             rows0, rows1, rows2, rows3, sidx0, sidx1, sidx2, sidx3,
             didx0, didx1, didx2, didx3, vals0, vals1, vals2, vals3, ones,
             sin0, sin1, sin2, sin3, ssc0, ssc1, ssc2, ssc3, sg):
    c = lax.axis_index("c")
    s = lax.axis_index("s")
    dh = d // _NC
    c0 = c * dh
    rows_pt = np_ // _NS
    r0 = s * rows_pt
    ep = e // _NS
    nch = ep // chunk
    e_base = s * ep
    nbuf = 4
    rows_ = (rows0, rows1, rows2, rows3)
    sidx_ = (sidx0, sidx1, sidx2, sidx3)
    didx_ = (didx0, didx1, didx2, didx3)
    vals_ = (vals0, vals1, vals2, vals3)
    sin_ = (sin0, sin1, sin2, sin3)
    ssc_ = (ssc0, ssc1, ssc2, ssc3)

    pltpu.sync_copy(z2, m0sh.at[pl.ds(r0, rows_pt)])
    pltpu.sync_copy(z2, m1sh.at[pl.ds(r0, rows_pt)])
    pltpu.sync_copy(ones_in, ones)

    @pl.when(s == 0)
    def _():
        pltpu.sync_copy(z1, ssh_a)
        pltpu.sync_copy(z1, ssh_b)
        pltpu.sync_copy(b2h, b2sh)

    plsc.subcore_barrier()

    def start_in(j, b):
        e0 = e_base + j * chunk
        pltpu.async_copy(ea.at[pl.ds(e0, chunk), pl.ds(c0, dh)], rows_[b], sin_[b])
        pltpu.async_copy(srci.at[pl.ds(e0, chunk)], sidx_[b], sin_[b])
        pltpu.async_copy(dsti.at[pl.ds(e0, chunk)], didx_[b], sin_[b])

    def wait_in(j, b):
        e0 = e_base + j * chunk
        pltpu.make_async_copy(ea.at[pl.ds(e0, chunk), pl.ds(c0, dh)], rows_[b], sin_[b]).wait()
        pltpu.make_async_copy(srci.at[pl.ds(e0, chunk)], sidx_[b], sin_[b]).wait()
        pltpu.make_async_copy(dsti.at[pl.ds(e0, chunk)], didx_[b], sin_[b]).wait()

    def start_scatter(b):
        pltpu.async_copy(rows_[b], m0sh.at[sidx_[b]], ssc_[b], add=True)
        pltpu.async_copy(rows_[b], m1sh.at[didx_[b]], ssc_[b], add=True)

        @pl.when(c == 0)
        def _():
            pltpu.async_copy(ones, ssh_a.at[sidx_[b]], ssc_[b], add=True)

        @pl.when(c == 1)
        def _():
            pltpu.async_copy(ones, ssh_a.at[didx_[b]], ssc_[b], add=True)

        @pl.when(c == (b % 2))
        def _():
            pltpu.async_copy(b2sh.at[didx_[b]], vals_[b], sg).wait()
            pltpu.async_copy(vals_[b], ssh_b.at[sidx_[b]], ssc_[b], add=True)

    def wait_scatter(b):
        pltpu.make_async_copy(rows_[b], m0sh.at[sidx_[b]], ssc_[b]).wait()
        pltpu.make_async_copy(rows_[b], m1sh.at[didx_[b]], ssc_[b]).wait()

        @pl.when(c == 0)
        def _():
            pltpu.make_async_copy(ones, ssh_a.at[sidx_[b]], ssc_[b]).wait()

        @pl.when(c == 1)
        def _():
            pltpu.make_async_copy(ones, ssh_a.at[didx_[b]], ssc_[b]).wait()

        @pl.when(c == (b % 2))
        def _():
            pltpu.make_async_copy(vals_[b], ssh_b.at[sidx_[b]], ssc_[b]).wait()

    start_in(0, 0)

    def _group(g, _):
        for b in range(nbuf):
            j = nbuf * g + b
            wait_in(j, b)
            start_scatter(b)

            @pl.when(j >= nbuf - 1)
            def _():
                wait_scatter((b + 1) % nbuf)

            @pl.when(j + 1 < nch)
            def _():
                start_in(j + 1, (b + 1) % nbuf)
        return ()

    lax.fori_loop(0, nch // nbuf, _group, ())

    tail = nch % nbuf
    base = nch - tail
    assert nch >= nbuf and tail in (0, 1)
    if tail == 1:
        wait_in(base, base % nbuf)
        start_scatter(base % nbuf)
        for b in range(nbuf):
            wait_scatter(b)
    else:
        for k in (1, 2, 3):
            wait_scatter((base - k) % nbuf)

    plsc.subcore_barrier()

    pltpu.sync_copy(m0sh.at[pl.ds(r0, rows_pt)],
                    m0o.at[pl.ds(r0, rows_pt), pl.ds(c0, dh)])
    pltpu.sync_copy(m1sh.at[pl.ds(r0, rows_pt)],
                    m1o.at[pl.ds(r0, rows_pt), pl.ds(c0, dh)])

    @pl.when(s == 0)
    def _():
        @pl.when(c == 0)
        def _():
            pltpu.sync_copy(ssh_a, cso)
            pltpu.sync_copy(ssh_b, t1ao)

        @pl.when(c == 1)
        def _():
            pltpu.sync_copy(ssh_a, cdo)
            pltpu.sync_copy(ssh_b, t1bo)


def _sc_call(edge_attr, src, dst, b2pad, np_, chunk):
    e, d = edge_attr.shape
    dh = d // _NC
    mesh = plsc.VectorSubcoreMesh(core_axis_name="c", subcore_axis_name="s")
    f32 = jnp.float32
    kern = pl.kernel(
        functools.partial(_sc_body, np_, e, d, chunk),
        out_type=[
            jax.ShapeDtypeStruct((np_, d), f32),
            jax.ShapeDtypeStruct((np_, d), f32),
            jax.ShapeDtypeStruct((np_,), f32),
            jax.ShapeDtypeStruct((np_,), f32),
            jax.ShapeDtypeStruct((np_,), f32),
            jax.ShapeDtypeStruct((np_,), f32),
        ],
        mesh=mesh,
        compiler_params=pltpu.CompilerParams(use_tc_tiling_on_sc=False),
        scratch_types=[
            pltpu.VMEM_SHARED((np_, dh), f32),
            pltpu.VMEM_SHARED((np_, dh), f32),
            pltpu.VMEM_SHARED((np_,), f32),
            pltpu.VMEM_SHARED((np_,), f32),
            pltpu.VMEM_SHARED((np_,), f32),
            *([pltpu.VMEM((chunk, dh), f32)] * 4),
            *([pltpu.VMEM((chunk,), jnp.int32)] * 4),
            *([pltpu.VMEM((chunk,), jnp.int32)] * 4),
            *([pltpu.VMEM((chunk,), f32)] * 4),
            pltpu.VMEM((chunk,), f32),
            *([pltpu.SemaphoreType.DMA] * 4),
            *([pltpu.SemaphoreType.DMA] * 4),
            pltpu.SemaphoreType.DMA,
        ],
    )
    z2 = jnp.zeros((np_ // _NS, dh), f32)
    z1 = jnp.zeros((np_,), f32)
    ones_in = jnp.ones((chunk,), f32)
    return kern(edge_attr, src, dst, b2pad, z2, z1, ones_in)


def _post_body(xt_ref, m0_ref, m1_ref, cs_ref, cd_ref, t1a_ref, t1b_ref,
               w1_ref, w3_ref, b_ref, o_ref):
    xt = xt_ref[...]
    m0s = m0_ref[...]
    m1s = m1_ref[...]
    cs = cs_ref[...]
    cd = cd_ref[...]
    a = jnp.sum(xt * w1_ref[...], axis=1, keepdims=True)
    e3 = jnp.sum(m0s * w3_ref[...], axis=1, keepdims=True)
    s_val = cs * (a + b_ref[0, 0]) + (t1a_ref[...] + t1b_ref[...]) + e3
    m0 = m0s / jnp.maximum(cs, 1.0)
    m1 = m1s / jnp.maximum(cd, 1.0)
    sup = xt * (1.0 + s_val) + m0 + m1
    un = jnp.maximum(jnp.sqrt(jnp.sum(sup * sup, axis=1, keepdims=True)), 1e-15)
    ex = jnp.tanh(un) * sup / un
    en = jnp.maximum(jnp.sqrt(jnp.sum(ex * ex, axis=1, keepdims=True)), 1e-15)
    maxn = (1.0 - 4e-3) / (_C ** 0.5)
    o_ref[...] = jnp.where(en > maxn, ex / en * maxn, ex)


def _post(xt, m0sum, m1sum, cs, cd, t1a, t1b, w1row, w3row, bias, bn):
    n, d = xt.shape
    grid = (n // bn,)
    col = pl.BlockSpec((bn, 1), lambda i: (i, 0))
    mat = pl.BlockSpec((bn, d), lambda i: (i, 0))
    wsp = pl.BlockSpec((1, d), lambda i: (0, 0))
    return pl.pallas_call(
        _post_body,
        grid=grid,
        in_specs=[mat, mat, mat, col, col, col, col, wsp, wsp,
                  pl.BlockSpec((1, 1), lambda i: (0, 0))],
        out_specs=mat,
        out_shape=jax.ShapeDtypeStruct((n, d), jnp.float32),
    )(xt, m0sum, m1sum, cs, cd, t1a, t1b, w1row, w3row, bias)


def kernel(x, adj, edge_attr, att_w, att_b):
    n, d = x.shape
    e, de = edge_attr.shape
    w1row = att_w[0:d, 0].reshape(1, d)
    w2row = att_w[d:2 * d, 0].reshape(1, d)
    w3row = att_w[2 * d:2 * d + de, 0].reshape(1, de)
    bias = att_b.reshape(1, 1)
    src = adj[0]
    dst = adj[1]

    bn = 2000
    xt, b2 = _pre(x, w2row, bn)

    np_ = ((n + 8 * _NS - 1) // (8 * _NS)) * (8 * _NS)
    b2pad = jnp.pad(b2.reshape(n), (0, np_ - n))

    chunk = 160
    m0sum, m1sum, cs, cd, t1a, t1b = _sc_call(
        edge_attr, src, dst, b2pad, np_, chunk)

    return _post(xt, m0sum, m1sum, cs.reshape(np_, 1), cd.reshape(np_, 1),
                 t1a.reshape(np_, 1), t1b.reshape(np_, 1),
                 w1row, w3row, bias, bn)


# speedup vs baseline: 1.0253x; 1.0253x over previous
import functools

import jax
import jax.numpy as jnp
from jax import lax
from jax.experimental import pallas as pl
from jax.experimental.pallas import tpu as pltpu
from jax.experimental.pallas import tpu_sc as plsc

_C = 1.0
_NS = 16
_NC = 2


def _logmap0_scale(x):
    nrm = jnp.maximum(jnp.sqrt(jnp.sum(x * x, axis=1, keepdims=True)), 1e-15)
    cn = jnp.clip(nrm, -1.0 + 1e-7, 1.0 - 1e-7)
    artanh = 0.5 * (jnp.log1p(cn) - jnp.log1p(-cn))
    return artanh / nrm


def _pre_body(x_ref, w2_ref, b2_ref):
    x = x_ref[...]
    xt = _logmap0_scale(x) * x
    b2_ref[...] = jnp.sum(xt * w2_ref[...], axis=1, keepdims=True)


def _pre(x, w2row, bn):
    n, d = x.shape
    grid = (n // bn,)
    return pl.pallas_call(
        _pre_body,
        grid=grid,
        in_specs=[
            pl.BlockSpec((bn, d), lambda i: (i, 0)),
            pl.BlockSpec((1, d), lambda i: (0, 0)),
        ],
        out_specs=pl.BlockSpec((bn, 1), lambda i: (i, 0)),
        out_shape=jax.ShapeDtypeStruct((n, 1), jnp.float32),
    )(x, w2row)


def _sc_body(np_, e, d, chunk,
             ea, srci, dsti, b2h, z2, z1, ones_in,
             m0o, m1o, cso, cdo, t1ao, t1bo,
             m0sh, m1sh, ssh_a, ssh_b, b2sh,
             rows0, rows1, sidx0, sidx1, didx0, didx1, vals0, vals1, ones,
             sin0, sin1, ssc0, ssc1, sg):
    c = lax.axis_index("c")
    s = lax.axis_index("s")
    dh = d // _NC
    c0 = c * dh
    rows_pt = np_ // _NS
    r0 = s * rows_pt
    ep = e // _NS
    nch = ep // chunk
    e_base = s * ep
    nbuf = 2
    rows_ = (rows0, rows1)
    sidx_ = (sidx0, sidx1)
    didx_ = (didx0, didx1)
    vals_ = (vals0, vals1)
    sin_ = (sin0, sin1)
    ssc_ = (ssc0, ssc1)

    pltpu.sync_copy(z2, m0sh.at[pl.ds(r0, rows_pt)])
    pltpu.sync_copy(z2, m1sh.at[pl.ds(r0, rows_pt)])
    pltpu.sync_copy(ones_in, ones)

    @pl.when(s == 0)
    def _():
        pltpu.sync_copy(z1, ssh_a)
        pltpu.sync_copy(z1, ssh_b)
        pltpu.sync_copy(b2h, b2sh)

    plsc.subcore_barrier()

    def start_in(j, b):
        e0 = e_base + j * chunk
        pltpu.async_copy(ea.at[pl.ds(e0, chunk), pl.ds(c0, dh)], rows_[b], sin_[b])
        pltpu.async_copy(srci.at[pl.ds(e0, chunk)], sidx_[b], sin_[b])
        pltpu.async_copy(dsti.at[pl.ds(e0, chunk)], didx_[b], sin_[b])

    def wait_in(j, b):
        e0 = e_base + j * chunk
        pltpu.make_async_copy(ea.at[pl.ds(e0, chunk), pl.ds(c0, dh)], rows_[b], sin_[b]).wait()
        pltpu.make_async_copy(srci.at[pl.ds(e0, chunk)], sidx_[b], sin_[b]).wait()
        pltpu.make_async_copy(dsti.at[pl.ds(e0, chunk)], didx_[b], sin_[b]).wait()

    def start_scatter(b):
        pltpu.async_copy(rows_[b], m0sh.at[sidx_[b]], ssc_[b], add=True)
        pltpu.async_copy(rows_[b], m1sh.at[didx_[b]], ssc_[b], add=True)

        @pl.when(c == 0)
        def _():
            pltpu.async_copy(ones, ssh_a.at[sidx_[b]], ssc_[b], add=True)

        @pl.when(c == 1)
        def _():
            pltpu.async_copy(ones, ssh_a.at[didx_[b]], ssc_[b], add=True)

        @pl.when(c == (b % 2))
        def _():
            pltpu.async_copy(b2sh.at[didx_[b]], vals_[b], sg).wait()
            pltpu.async_copy(vals_[b], ssh_b.at[sidx_[b]], ssc_[b], add=True)

    def wait_scatter(b):
        pltpu.make_async_copy(rows_[b], m0sh.at[sidx_[b]], ssc_[b]).wait()
        pltpu.make_async_copy(rows_[b], m1sh.at[didx_[b]], ssc_[b]).wait()

        @pl.when(c == 0)
        def _():
            pltpu.make_async_copy(ones, ssh_a.at[sidx_[b]], ssc_[b]).wait()

        @pl.when(c == 1)
        def _():
            pltpu.make_async_copy(ones, ssh_a.at[didx_[b]], ssc_[b]).wait()

        @pl.when(c == (b % 2))
        def _():
            pltpu.make_async_copy(vals_[b], ssh_b.at[sidx_[b]], ssc_[b]).wait()

    start_in(0, 0)

    def _group(g, _):
        for b in range(nbuf):
            j = nbuf * g + b
            wait_in(j, b)
            start_scatter(b)

            @pl.when(j >= nbuf - 1)
            def _():
                wait_scatter((b + 1) % nbuf)

            @pl.when(j + 1 < nch)
            def _():
                start_in(j + 1, (b + 1) % nbuf)
        return ()

    lax.fori_loop(0, nch // nbuf, _group, ())

    tail = nch % nbuf
    base = nch - tail
    assert nch >= nbuf and tail in (0, 1)
    if tail == 1:
        wait_in(base, base % nbuf)
        start_scatter(base % nbuf)
        for b in range(nbuf):
            wait_scatter(b)
    else:
        for k in range(1, nbuf):
            wait_scatter((base - k) % nbuf)

    plsc.subcore_barrier()

    pltpu.sync_copy(m0sh.at[pl.ds(r0, rows_pt)],
                    m0o.at[pl.ds(r0, rows_pt), pl.ds(c0, dh)])
    pltpu.sync_copy(m1sh.at[pl.ds(r0, rows_pt)],
                    m1o.at[pl.ds(r0, rows_pt), pl.ds(c0, dh)])

    @pl.when(s == 0)
    def _():
        @pl.when(c == 0)
        def _():
            pltpu.sync_copy(ssh_a, cso)
            pltpu.sync_copy(ssh_b, t1ao)

        @pl.when(c == 1)
        def _():
            pltpu.sync_copy(ssh_a, cdo)
            pltpu.sync_copy(ssh_b, t1bo)


def _sc_call(edge_attr, src, dst, b2pad, np_, chunk):
    e, d = edge_attr.shape
    dh = d // _NC
    mesh = plsc.VectorSubcoreMesh(core_axis_name="c", subcore_axis_name="s")
    f32 = jnp.float32
    kern = pl.kernel(
        functools.partial(_sc_body, np_, e, d, chunk),
        out_type=[
            jax.ShapeDtypeStruct((np_, d), f32),
            jax.ShapeDtypeStruct((np_, d), f32),
            jax.ShapeDtypeStruct((np_,), f32),
            jax.ShapeDtypeStruct((np_,), f32),
            jax.ShapeDtypeStruct((np_,), f32),
            jax.ShapeDtypeStruct((np_,), f32),
        ],
        mesh=mesh,
        compiler_params=pltpu.CompilerParams(use_tc_tiling_on_sc=False),
        scratch_types=[
            pltpu.VMEM_SHARED((np_, dh), f32),
            pltpu.VMEM_SHARED((np_, dh), f32),
            pltpu.VMEM_SHARED((np_,), f32),
            pltpu.VMEM_SHARED((np_,), f32),
            pltpu.VMEM_SHARED((np_,), f32),
            *([pltpu.VMEM((chunk, dh), f32)] * 2),
            *([pltpu.VMEM((chunk,), jnp.int32)] * 2),
            *([pltpu.VMEM((chunk,), jnp.int32)] * 2),
            *([pltpu.VMEM((chunk,), f32)] * 2),
            pltpu.VMEM((chunk,), f32),
            *([pltpu.SemaphoreType.DMA] * 2),
            *([pltpu.SemaphoreType.DMA] * 2),
            pltpu.SemaphoreType.DMA,
        ],
    )
    z2 = jnp.zeros((np_ // _NS, dh), f32)
    z1 = jnp.zeros((np_,), f32)
    ones_in = jnp.ones((chunk,), f32)
    return kern(edge_attr, src, dst, b2pad, z2, z1, ones_in)


def _post_body(x_ref, m0_ref, m1_ref, cs_ref, cd_ref, t1a_ref, t1b_ref,
               w1_ref, w3_ref, b_ref, o_ref):
    x = x_ref[...]
    xt = _logmap0_scale(x) * x
    m0s = m0_ref[...]
    m1s = m1_ref[...]
    cs = cs_ref[...]
    cd = cd_ref[...]
    a = jnp.sum(xt * w1_ref[...], axis=1, keepdims=True)
    e3 = jnp.sum(m0s * w3_ref[...], axis=1, keepdims=True)
    s_val = cs * (a + b_ref[0, 0]) + (t1a_ref[...] + t1b_ref[...]) + e3
    m0 = m0s / jnp.maximum(cs, 1.0)
    m1 = m1s / jnp.maximum(cd, 1.0)
    sup = xt * (1.0 + s_val) + m0 + m1
    un = jnp.maximum(jnp.sqrt(jnp.sum(sup * sup, axis=1, keepdims=True)), 1e-15)
    ex = jnp.tanh(un) * sup / un
    en = jnp.maximum(jnp.sqrt(jnp.sum(ex * ex, axis=1, keepdims=True)), 1e-15)
    maxn = (1.0 - 4e-3) / (_C ** 0.5)
    o_ref[...] = jnp.where(en > maxn, ex / en * maxn, ex)


def _post(x, m0sum, m1sum, cs, cd, t1a, t1b, w1row, w3row, bias, bn):
    n, d = x.shape
    grid = (n // bn,)
    col = pl.BlockSpec((bn, 1), lambda i: (i, 0))
    mat = pl.BlockSpec((bn, d), lambda i: (i, 0))
    wsp = pl.BlockSpec((1, d), lambda i: (0, 0))
    return pl.pallas_call(
        _post_body,
        grid=grid,
        in_specs=[mat, mat, mat, col, col, col, col, wsp, wsp,
                  pl.BlockSpec((1, 1), lambda i: (0, 0))],
        out_specs=mat,
        out_shape=jax.ShapeDtypeStruct((n, d), jnp.float32),
    )(x, m0sum, m1sum, cs, cd, t1a, t1b, w1row, w3row, bias)


def kernel(x, adj, edge_attr, att_w, att_b):
    n, d = x.shape
    e, de = edge_attr.shape
    w1row = att_w[0:d, 0].reshape(1, d)
    w2row = att_w[d:2 * d, 0].reshape(1, d)
    w3row = att_w[2 * d:2 * d + de, 0].reshape(1, de)
    bias = att_b.reshape(1, 1)
    src = adj[0]
    dst = adj[1]

    bn = 2000
    b2 = _pre(x, w2row, bn)

    np_ = ((n + 8 * _NS - 1) // (8 * _NS)) * (8 * _NS)
    b2pad = jnp.pad(b2.reshape(n), (0, np_ - n))

    chunk = 200
    m0sum, m1sum, cs, cd, t1a, t1b = _sc_call(
        edge_attr, src, dst, b2pad, np_, chunk)

    return _post(x, m0sum, m1sum, cs.reshape(np_, 1), cd.reshape(np_, 1),
                 t1a.reshape(np_, 1), t1b.reshape(np_, 1),
                 w1row, w3row, bias, bn)


# speedup vs baseline: 1.0699x; 1.0435x over previous
import functools

import jax
import jax.numpy as jnp
from jax import lax
from jax.experimental import pallas as pl
from jax.experimental.pallas import tpu as pltpu
from jax.experimental.pallas import tpu_sc as plsc

_C = 1.0
_NS = 16
_NC = 2


def _logmap0_scale(x):
    nrm = jnp.maximum(jnp.sqrt(jnp.sum(x * x, axis=1, keepdims=True)), 1e-15)
    cn = jnp.clip(nrm, -1.0 + 1e-7, 1.0 - 1e-7)
    artanh = 0.5 * (jnp.log1p(cn) - jnp.log1p(-cn))
    return artanh / nrm


def _pre_body(x_ref, w2_ref, b2_ref):
    x = x_ref[...]
    xt = _logmap0_scale(x) * x
    b2_ref[...] = jnp.sum(xt * w2_ref[...], axis=1, keepdims=True)


def _pre(x, w2row, bn):
    n, d = x.shape
    grid = (n // bn,)
    return pl.pallas_call(
        _pre_body,
        grid=grid,
        in_specs=[
            pl.BlockSpec((bn, d), lambda i: (i, 0)),
            pl.BlockSpec((1, d), lambda i: (0, 0)),
        ],
        out_specs=pl.BlockSpec((bn, 1), lambda i: (i, 0)),
        out_shape=jax.ShapeDtypeStruct((n, 1), jnp.float32),
    )(x, w2row)


def _sc_body(np_, e, d, chunk,
             ea, adj, b2h, z2, z1, ones_in,
             m0o, m1o, cso, cdo, t1ao, t1bo,
             m0sh, m1sh, ssh_a, ssh_b, b2sh,
             rows0, rows1, sidx0, sidx1, didx0, didx1, vals0, vals1, ones,
             sin0, sin1, ssc0, ssc1, sg):
    c = lax.axis_index("c")
    s = lax.axis_index("s")
    dh = d // _NC
    c0 = c * dh
    rows_pt = np_ // _NS
    r0 = s * rows_pt
    ep = e // _NS
    nch = ep // chunk
    e_base = s * ep
    nbuf = 2
    rows_ = (rows0, rows1)
    sidx_ = (sidx0, sidx1)
    didx_ = (didx0, didx1)
    vals_ = (vals0, vals1)
    sin_ = (sin0, sin1)
    ssc_ = (ssc0, ssc1)

    pltpu.sync_copy(z2, m0sh.at[pl.ds(r0, rows_pt)])
    pltpu.sync_copy(z2, m1sh.at[pl.ds(r0, rows_pt)])
    pltpu.sync_copy(ones_in, ones)

    @pl.when(s == 0)
    def _():
        pltpu.sync_copy(z1, ssh_a)
        pltpu.sync_copy(z1, ssh_b)
        pltpu.sync_copy(b2h, b2sh)

    plsc.subcore_barrier()

    def start_in(j, b):
        e0 = e_base + j * chunk
        pltpu.async_copy(ea.at[pl.ds(e0, chunk), pl.ds(c0, dh)], rows_[b], sin_[b])
        pltpu.async_copy(adj.at[0, pl.ds(e0, chunk)], sidx_[b], sin_[b])
        pltpu.async_copy(adj.at[1, pl.ds(e0, chunk)], didx_[b], sin_[b])

    def wait_in(j, b):
        e0 = e_base + j * chunk
        pltpu.make_async_copy(ea.at[pl.ds(e0, chunk), pl.ds(c0, dh)], rows_[b], sin_[b]).wait()
        pltpu.make_async_copy(adj.at[0, pl.ds(e0, chunk)], sidx_[b], sin_[b]).wait()
        pltpu.make_async_copy(adj.at[1, pl.ds(e0, chunk)], didx_[b], sin_[b]).wait()

    def start_scatter(b):
        pltpu.async_copy(rows_[b], m0sh.at[sidx_[b]], ssc_[b], add=True)
        pltpu.async_copy(rows_[b], m1sh.at[didx_[b]], ssc_[b], add=True)

        @pl.when(c == 0)
        def _():
            pltpu.async_copy(ones, ssh_a.at[sidx_[b]], ssc_[b], add=True)

        @pl.when(c == 1)
        def _():
            pltpu.async_copy(ones, ssh_a.at[didx_[b]], ssc_[b], add=True)

        @pl.when(c == (b % 2))
        def _():
            pltpu.async_copy(b2sh.at[didx_[b]], vals_[b], sg).wait()
            pltpu.async_copy(vals_[b], ssh_b.at[sidx_[b]], ssc_[b], add=True)

    def wait_scatter(b):
        pltpu.make_async_copy(rows_[b], m0sh.at[sidx_[b]], ssc_[b]).wait()
        pltpu.make_async_copy(rows_[b], m1sh.at[didx_[b]], ssc_[b]).wait()

        @pl.when(c == 0)
        def _():
            pltpu.make_async_copy(ones, ssh_a.at[sidx_[b]], ssc_[b]).wait()

        @pl.when(c == 1)
        def _():
            pltpu.make_async_copy(ones, ssh_a.at[didx_[b]], ssc_[b]).wait()

        @pl.when(c == (b % 2))
        def _():
            pltpu.make_async_copy(vals_[b], ssh_b.at[sidx_[b]], ssc_[b]).wait()

    start_in(0, 0)

    def _group(g, _):
        for b in range(nbuf):
            j = nbuf * g + b
            wait_in(j, b)
            start_scatter(b)

            @pl.when(j >= nbuf - 1)
            def _():
                wait_scatter((b + 1) % nbuf)

            @pl.when(j + 1 < nch)
            def _():
                start_in(j + 1, (b + 1) % nbuf)
        return ()

    lax.fori_loop(0, nch // nbuf, _group, ())

    tail = nch % nbuf
    base = nch - tail
    assert nch >= nbuf and tail in (0, 1)
    if tail == 1:
        wait_in(base, base % nbuf)
        start_scatter(base % nbuf)
        for b in range(nbuf):
            wait_scatter(b)
    else:
        for k in range(1, nbuf):
            wait_scatter((base - k) % nbuf)

    plsc.subcore_barrier()

    pltpu.sync_copy(m0sh.at[pl.ds(r0, rows_pt)],
                    m0o.at[pl.ds(r0, rows_pt), pl.ds(c0, dh)])
    pltpu.sync_copy(m1sh.at[pl.ds(r0, rows_pt)],
                    m1o.at[pl.ds(r0, rows_pt), pl.ds(c0, dh)])

    @pl.when(s == 0)
    def _():
        @pl.when(c == 0)
        def _():
            pltpu.sync_copy(ssh_a, cso)
            pltpu.sync_copy(ssh_b, t1ao)

        @pl.when(c == 1)
        def _():
            pltpu.sync_copy(ssh_a, cdo)
            pltpu.sync_copy(ssh_b, t1bo)


def _sc_call(edge_attr, adj, b2pad, np_, chunk):
    e, d = edge_attr.shape
    dh = d // _NC
    mesh = plsc.VectorSubcoreMesh(core_axis_name="c", subcore_axis_name="s")
    f32 = jnp.float32
    kern = pl.kernel(
        functools.partial(_sc_body, np_, e, d, chunk),
        out_type=[
            jax.ShapeDtypeStruct((np_, d), f32),
            jax.ShapeDtypeStruct((np_, d), f32),
            jax.ShapeDtypeStruct((np_,), f32),
            jax.ShapeDtypeStruct((np_,), f32),
            jax.ShapeDtypeStruct((np_,), f32),
            jax.ShapeDtypeStruct((np_,), f32),
        ],
        mesh=mesh,
        compiler_params=pltpu.CompilerParams(use_tc_tiling_on_sc=False),
        scratch_types=[
            pltpu.VMEM_SHARED((np_, dh), f32),
            pltpu.VMEM_SHARED((np_, dh), f32),
            pltpu.VMEM_SHARED((np_,), f32),
            pltpu.VMEM_SHARED((np_,), f32),
            pltpu.VMEM_SHARED((np_,), f32),
            *([pltpu.VMEM((chunk, dh), f32)] * 2),
            *([pltpu.VMEM((chunk,), jnp.int32)] * 2),
            *([pltpu.VMEM((chunk,), jnp.int32)] * 2),
            *([pltpu.VMEM((chunk,), f32)] * 2),
            pltpu.VMEM((chunk,), f32),
            *([pltpu.SemaphoreType.DMA] * 2),
            *([pltpu.SemaphoreType.DMA] * 2),
            pltpu.SemaphoreType.DMA,
        ],
    )
    z2 = jnp.zeros((np_ // _NS, dh), f32)
    z1 = jnp.zeros((np_,), f32)
    ones_in = jnp.ones((chunk,), f32)
    return kern(edge_attr, adj, b2pad, z2, z1, ones_in)


def _post_body(x_ref, m0_ref, m1_ref, cs_ref, cd_ref, t1a_ref, t1b_ref,
               w1_ref, w3_ref, b_ref, o_ref):
    x = x_ref[...]
    xt = _logmap0_scale(x) * x
    m0s = m0_ref[...]
    m1s = m1_ref[...]
    cs = cs_ref[...]
    cd = cd_ref[...]
    a = jnp.sum(xt * w1_ref[...], axis=1, keepdims=True)
    e3 = jnp.sum(m0s * w3_ref[...], axis=1, keepdims=True)
    s_val = cs * (a + b_ref[0, 0]) + (t1a_ref[...] + t1b_ref[...]) + e3
    m0 = m0s / jnp.maximum(cs, 1.0)
    m1 = m1s / jnp.maximum(cd, 1.0)
    sup = xt * (1.0 + s_val) + m0 + m1
    un = jnp.maximum(jnp.sqrt(jnp.sum(sup * sup, axis=1, keepdims=True)), 1e-15)
    ex = jnp.tanh(un) * sup / un
    en = jnp.maximum(jnp.sqrt(jnp.sum(ex * ex, axis=1, keepdims=True)), 1e-15)
    maxn = (1.0 - 4e-3) / (_C ** 0.5)
    o_ref[...] = jnp.where(en > maxn, ex / en * maxn, ex)


def _post(x, m0sum, m1sum, cs, cd, t1a, t1b, w1row, w3row, bias, bn):
    n, d = x.shape
    grid = (n // bn,)
    col = pl.BlockSpec((bn, 1), lambda i: (i, 0))
    mat = pl.BlockSpec((bn, d), lambda i: (i, 0))
    wsp = pl.BlockSpec((1, d), lambda i: (0, 0))
    return pl.pallas_call(
        _post_body,
        grid=grid,
        in_specs=[mat, mat, mat, col, col, col, col, wsp, wsp,
                  pl.BlockSpec((1, 1), lambda i: (0, 0))],
        out_specs=mat,
        out_shape=jax.ShapeDtypeStruct((n, d), jnp.float32),
    )(x, m0sum, m1sum, cs, cd, t1a, t1b, w1row, w3row, bias)


def kernel(x, adj, edge_attr, att_w, att_b):
    n, d = x.shape
    e, de = edge_attr.shape
    w1row = att_w[0:d, 0].reshape(1, d)
    w2row = att_w[d:2 * d, 0].reshape(1, d)
    w3row = att_w[2 * d:2 * d + de, 0].reshape(1, de)
    bias = att_b.reshape(1, 1)

    np_ = ((n + 8 * _NS - 1) // (8 * _NS)) * (8 * _NS)

    bn = 2000
    b2 = _pre(x, w2row, bn)
    b2pad = jnp.pad(b2.reshape(n), (0, np_ - n))

    chunk = 200
    m0sum, m1sum, cs, cd, t1a, t1b = _sc_call(edge_attr, adj, b2pad, np_, chunk)

    return _post(x, m0sum, m1sum, cs.reshape(np_, 1), cd.reshape(np_, 1),
                 t1a.reshape(np_, 1), t1b.reshape(np_, 1),
                 w1row, w3row, bias, bn)
